# baseline (speedup 1.0000x reference)
import functools

import jax
import jax.numpy as jnp
from jax import lax
from jax.experimental import pallas as pl
from jax.experimental.pallas import tpu as pltpu
from jax.experimental.pallas import tpu_sc as plsc

N_ROWS = 9216
N_CODES = 1024
DIM = 64
BLOCK_ROWS = 512
N_BLOCKS = N_ROWS // BLOCK_ROWS


def _norm_rows(s):
    acc = s[:, 0:8]
    for c in range(1, 8):
        acc = acc + s[:, 8 * c:8 * c + 8]
    t = acc[:, 0:4] + acc[:, 4:8]
    t = t[:, 0:2] + t[:, 2:4]
    return t[:, 0:1] + t[:, 1:2]


def _norm_cols(s):
    acc = s[0:8, :]
    for c in range(1, 8):
        acc = acc + s[8 * c:8 * c + 8, :]
    t = acc[0:4, :] + acc[4:8, :]
    t = t[0:2, :] + t[2:4, :]
    return t[0:1, :] + t[1:2, :]


def _argmin_body(z_ref, w_ref, wt_ref, idx_ref, loss_ref):
    z = z_ref[...]
    w = w_ref[...]
    wt = wt_ref[...]
    x2 = _norm_rows(z * z)
    y2 = _norm_cols(wt * wt)
    mm = lax.dot_general(z, w, (((1,), (1,)), ((), ())),
                         preferred_element_type=jnp.float32)
    d2 = x2 + y2 - 2.0 * mm
    d = jnp.sqrt(jnp.maximum(d2, 0.0))
    dmin = jnp.min(d, axis=1, keepdims=True)
    k_iota = lax.broadcasted_iota(jnp.int32, d.shape, 1)
    idx = jnp.min(jnp.where(d == dmin, k_iota, jnp.int32(2**30)), axis=1)
    idx_ref[0, 0, :] = idx

    @pl.when(pl.program_id(0) == 0)
    def _():
        loss_ref[...] = jnp.zeros((1, 1), jnp.float32)

    loss_ref[...] += jnp.sum(dmin * dmin, axis=0, keepdims=True)


def _tc_argmin(z_flat, weight, weight_t):
    return pl.pallas_call(
        _argmin_body,
        grid=(N_BLOCKS,),
        in_specs=[
            pl.BlockSpec((BLOCK_ROWS, DIM), lambda i: (i, 0)),
            pl.BlockSpec((N_CODES, DIM), lambda i: (0, 0)),
            pl.BlockSpec((DIM, N_CODES), lambda i: (0, 0)),
        ],
        out_specs=[
            pl.BlockSpec((1, 1, BLOCK_ROWS), lambda i: (i, 0, 0)),
            pl.BlockSpec((1, 1), lambda i: (0, 0)),
        ],
        out_shape=[
            jax.ShapeDtypeStruct((N_BLOCKS, 1, BLOCK_ROWS), jnp.int32),
            jax.ShapeDtypeStruct((1, 1), jnp.float32),
        ],
    )(z_flat, weight, weight_t)


_SC_CORES = 2
_SC_SUBCORES = 16
_NW = _SC_CORES * _SC_SUBCORES
_B_PER_W = N_ROWS // _NW


_GDIM = 128


def _sc_gather_body(table_hbm, idx_hbm, out_hbm, idx_v, rows_v, sem):
    wid = lax.axis_index("s") * _SC_CORES + lax.axis_index("c")
    base = wid * _B_PER_W
    pltpu.sync_copy(idx_hbm.at[pl.ds(base, _B_PER_W)], idx_v)
    pltpu.async_copy(table_hbm.at[idx_v], rows_v, sem).wait()
    pltpu.sync_copy(rows_v, out_hbm.at[pl.ds(base, _B_PER_W)])


def _sc_gather(table_padded, idx):
    mesh = plsc.VectorSubcoreMesh(core_axis_name="c", subcore_axis_name="s")
    fn = pl.kernel(
        _sc_gather_body,
        out_type=jax.ShapeDtypeStruct((N_ROWS, _GDIM), jnp.float32),
        mesh=mesh,
        scratch_types=[
            pltpu.VMEM((_B_PER_W,), jnp.int32),
            pltpu.VMEM((_B_PER_W, _GDIM), jnp.float32),
            pltpu.SemaphoreType.DMA,
        ],
    )
    return fn(table_padded, idx)


def kernel(z_e, weight):
    z_flat = z_e.reshape(-1, DIM)
    idx3, loss_num = _tc_argmin(z_flat, weight, weight.T)
    idx = idx3.reshape(N_ROWS)
    table_padded = jnp.pad(weight, ((0, 0), (0, _GDIM - DIM)))
    z_q = _sc_gather(table_padded, idx)[:, :DIM].reshape(z_e.shape)
    commitment_loss = loss_num[0, 0] / jnp.float32(N_ROWS * DIM)
    return (z_q, commitment_loss)


# speedup vs baseline: 1.0195x; 1.0195x over previous
import functools

import jax
import jax.numpy as jnp
from jax import lax
from jax.experimental import pallas as pl
from jax.experimental.pallas import tpu as pltpu
from jax.experimental.pallas import tpu_sc as plsc

N_ROWS = 9216
N_CODES = 1024
DIM = 64
BLOCK_ROWS = 512
N_BLOCKS = N_ROWS // BLOCK_ROWS


def _norm_rows(s):
    acc = s[:, 0:8]
    for c in range(1, 8):
        acc = acc + s[:, 8 * c:8 * c + 8]
    t = acc[:, 0:4] + acc[:, 4:8]
    t = t[:, 0:2] + t[:, 2:4]
    return t[:, 0:1] + t[:, 1:2]


def _norm_cols(s):
    acc = s[0:8, :]
    for c in range(1, 8):
        acc = acc + s[8 * c:8 * c + 8, :]
    t = acc[0:4, :] + acc[4:8, :]
    t = t[0:2, :] + t[2:4, :]
    return t[0:1, :] + t[1:2, :]


def _argmin_body(z_ref, w_ref, wt_ref, idx_ref, loss_ref):
    z = z_ref[...]
    w = w_ref[...]
    wt = wt_ref[...]
    x2 = _norm_rows(z * z)
    y2 = _norm_cols(wt * wt)
    mm = lax.dot_general(z, w, (((1,), (1,)), ((), ())),
                         preferred_element_type=jnp.float32)
    d2 = x2 + y2 - 2.0 * mm
    d = jnp.sqrt(jnp.maximum(d2, 0.0))
    dmin = jnp.min(d, axis=1, keepdims=True)
    k_iota = lax.broadcasted_iota(jnp.int32, d.shape, 1)
    idx = jnp.min(jnp.where(d == dmin, k_iota, jnp.int32(2**30)), axis=1)
    idx_ref[0, 0, :] = idx
    loss_ref[...] = jnp.sum(dmin * dmin, axis=0, keepdims=True)[None]


def _tc_argmin(z_flat, weight, weight_t):
    return pl.pallas_call(
        _argmin_body,
        grid=(N_BLOCKS,),
        in_specs=[
            pl.BlockSpec((BLOCK_ROWS, DIM), lambda i: (i, 0)),
            pl.BlockSpec((N_CODES, DIM), lambda i: (0, 0)),
            pl.BlockSpec((DIM, N_CODES), lambda i: (0, 0)),
        ],
        out_specs=[
            pl.BlockSpec((1, 1, BLOCK_ROWS), lambda i: (i, 0, 0)),
            pl.BlockSpec((1, 1, 1), lambda i: (i, 0, 0)),
        ],
        out_shape=[
            jax.ShapeDtypeStruct((N_BLOCKS, 1, BLOCK_ROWS), jnp.int32),
            jax.ShapeDtypeStruct((N_BLOCKS, 1, 1), jnp.float32),
        ],
        compiler_params=pltpu.CompilerParams(
            dimension_semantics=["parallel"]),
    )(z_flat, weight, weight_t)


_SC_CORES = 2
_SC_SUBCORES = 16
_NW = _SC_CORES * _SC_SUBCORES
_B_PER_W = N_ROWS // _NW


_GDIM = 128


def _sc_gather_body(table_hbm, idx_hbm, out_hbm, idx_v, rows_v, sem):
    wid = lax.axis_index("s") * _SC_CORES + lax.axis_index("c")
    base = wid * _B_PER_W
    pltpu.sync_copy(idx_hbm.at[pl.ds(base, _B_PER_W)], idx_v)
    pltpu.async_copy(table_hbm.at[idx_v], rows_v, sem).wait()
    pltpu.sync_copy(rows_v, out_hbm.at[pl.ds(base, _B_PER_W)])


def _sc_gather(table_padded, idx):
    mesh = plsc.VectorSubcoreMesh(core_axis_name="c", subcore_axis_name="s")
    fn = pl.kernel(
        _sc_gather_body,
        out_type=jax.ShapeDtypeStruct((N_ROWS, _GDIM), jnp.float32),
        mesh=mesh,
        scratch_types=[
            pltpu.VMEM((_B_PER_W,), jnp.int32),
            pltpu.VMEM((_B_PER_W, _GDIM), jnp.float32),
            pltpu.SemaphoreType.DMA,
        ],
    )
    return fn(table_padded, idx)


def kernel(z_e, weight):
    z_flat = z_e.reshape(-1, DIM)
    idx3, loss_num = _tc_argmin(z_flat, weight, weight.T)
    idx = idx3.reshape(N_ROWS)
    table_padded = jnp.pad(weight, ((0, 0), (0, _GDIM - DIM)))
    z_q = _sc_gather(table_padded, idx)[:, :DIM].reshape(z_e.shape)
    commitment_loss = jnp.sum(loss_num) / jnp.float32(N_ROWS * DIM)
    return (z_q, commitment_loss)


# speedup vs baseline: 1.1449x; 1.1231x over previous
import functools

import jax
import jax.numpy as jnp
from jax import lax
from jax.experimental import pallas as pl
from jax.experimental.pallas import tpu as pltpu
from jax.experimental.pallas import tpu_sc as plsc

N_ROWS = 9216
N_CODES = 1024
DIM = 64
BLOCK_ROWS = 512
N_BLOCKS = N_ROWS // BLOCK_ROWS


def _norm_cols(s):
    acc = s[0:8, :]
    for c in range(1, 8):
        acc = acc + s[8 * c:8 * c + 8, :]
    t = acc[0:4, :] + acc[4:8, :]
    t = t[0:2, :] + t[2:4, :]
    return t[0:1, :] + t[1:2, :]


def _argmin_body(z_ref, w_ref, wt_ref, zt_ref, idx_ref, loss_ref):
    z = z_ref[...]
    w = w_ref[...]
    wt = wt_ref[...]
    zt = zt_ref[...]
    x2 = lax.transpose(_norm_cols(zt * zt), (1, 0))
    y2 = _norm_cols(wt * wt)
    mm = lax.dot_general(z * (-2.0), w, (((1,), (1,)), ((), ())),
                         preferred_element_type=jnp.float32)
    d2 = (x2 + y2) + mm
    d = jnp.sqrt(jnp.maximum(d2, 0.0))
    dmin = jnp.min(d, axis=1, keepdims=True)
    k_iota = lax.broadcasted_iota(jnp.int32, d.shape, 1)
    k_f = lax.bitcast_convert_type(k_iota | jnp.int32(0x3F800000), jnp.float32)
    m = jnp.min(jnp.where(d == dmin, k_f, jnp.float32(2.0)), axis=1)
    idx = lax.bitcast_convert_type(m, jnp.int32) - jnp.int32(0x3F800000)
    idx_ref[0, 0, :] = idx
    loss_ref[...] = jnp.sum(dmin * dmin, axis=0, keepdims=True)[None]


def _tc_argmin(z_flat, weight, weight_t, z_t):
    return pl.pallas_call(
        _argmin_body,
        grid=(N_BLOCKS,),
        in_specs=[
            pl.BlockSpec((BLOCK_ROWS, DIM), lambda i: (i, 0)),
            pl.BlockSpec((N_CODES, DIM), lambda i: (0, 0)),
            pl.BlockSpec((DIM, N_CODES), lambda i: (0, 0)),
            pl.BlockSpec((DIM, BLOCK_ROWS), lambda i: (0, i)),
        ],
        out_specs=[
            pl.BlockSpec((1, 1, BLOCK_ROWS), lambda i: (i, 0, 0)),
            pl.BlockSpec((1, 1, 1), lambda i: (i, 0, 0)),
        ],
        out_shape=[
            jax.ShapeDtypeStruct((N_BLOCKS, 1, BLOCK_ROWS), jnp.int32),
            jax.ShapeDtypeStruct((N_BLOCKS, 1, 1), jnp.float32),
        ],
        compiler_params=pltpu.CompilerParams(
            dimension_semantics=["parallel"]),
    )(z_flat, weight, weight_t, z_t)


_SC_CORES = 2
_SC_SUBCORES = 16
_NW = _SC_CORES * _SC_SUBCORES
_B_PER_W = N_ROWS // _NW


_GDIM = 128


def _sc_gather_body(table_hbm, idx_hbm, out_hbm, idx_v, rows_v, sem):
    wid = lax.axis_index("s") * _SC_CORES + lax.axis_index("c")
    base = wid * _B_PER_W
    pltpu.sync_copy(idx_hbm.at[pl.ds(base, _B_PER_W)], idx_v)
    pltpu.async_copy(table_hbm.at[idx_v], rows_v, sem).wait()
    pltpu.sync_copy(rows_v, out_hbm.at[pl.ds(base, _B_PER_W)])


def _sc_gather(table_padded, idx):
    mesh = plsc.VectorSubcoreMesh(core_axis_name="c", subcore_axis_name="s")
    fn = pl.kernel(
        _sc_gather_body,
        out_type=jax.ShapeDtypeStruct((N_ROWS, _GDIM), jnp.float32),
        mesh=mesh,
        scratch_types=[
            pltpu.VMEM((_B_PER_W,), jnp.int32),
            pltpu.VMEM((_B_PER_W, _GDIM), jnp.float32),
            pltpu.SemaphoreType.DMA,
        ],
    )
    return fn(table_padded, idx)


def kernel(z_e, weight):
    z_flat = z_e.reshape(-1, DIM)
    idx3, loss_num = _tc_argmin(z_flat, weight, weight.T, z_flat.T)
    idx = idx3.reshape(N_ROWS)
    table_padded = jnp.pad(weight, ((0, 0), (0, _GDIM - DIM)))
    z_q = _sc_gather(table_padded, idx)[:, :DIM].reshape(z_e.shape)
    commitment_loss = jnp.sum(loss_num) / jnp.float32(N_ROWS * DIM)
    return (z_q, commitment_loss)
